# baseline (speedup 1.0000x reference)
import math

import jax
import jax.numpy as jnp
from jax import lax
from jax.experimental import pallas as pl
from jax.experimental.pallas import tpu as pltpu
from jax.experimental.pallas import tpu_sc as plsc

D_MODEL = 64
LANES = 16
NC = 2
NS = 16
NW = NC * NS
CHUNK = 128
GPC = 2
SUP = CHUNK * GPC
SCALE = math.sqrt(D_MODEL)


def _body(table, toks, bias, out, idx_v, qb0, qb1, ob0, ob1, bias_v,
          gsem0, gsem1, osem0, osem1):
    n_chunks = toks.shape[1]
    n_sup = n_chunks // GPC
    n_pair = n_sup // 2
    qbs = (qb0, qb1)
    obs = (ob0, ob1)
    gsems = (gsem0, gsem1)
    osems = (osem0, osem1)
    wid = lax.axis_index("s") * NC + lax.axis_index("c")
    base = wid * n_chunks * CHUNK

    pltpu.sync_copy(toks.at[wid], idx_v)
    pltpu.sync_copy(bias, bias_v)
    b_regs = [bias_v[pl.ds(k * LANES, LANES)] for k in range(D_MODEL // LANES)]

    def fire(j, b):
        for i in range(GPC):
            pltpu.async_copy(
                table.at[idx_v.at[j * GPC + i]],
                qbs[b].at[pl.ds(i * CHUNK, CHUNK)],
                gsems[b],
            )

    def drain(j, b):
        for i in range(GPC):
            pltpu.make_async_copy(
                table.at[idx_v.at[j * GPC + i]],
                qbs[b].at[pl.ds(i * CHUNK, CHUNK)],
                gsems[b],
            ).wait()

    def fire_pair(q, p):
        pltpu.async_copy(
            obs[p],
            out.at[pl.ds(base + q * 2 * SUP, 2 * SUP)],
            osems[p],
        )

    def wait_pair(q, p):
        pltpu.make_async_copy(
            obs[p],
            out.at[pl.ds(base + q * 2 * SUP, 2 * SUP)],
            osems[p],
        ).wait()

    fire(0, 0)

    @pl.loop(0, n_sup)
    def _sup(j):
        for b in range(2):

            @pl.when(j % 2 == b)
            def _():
                @pl.when(j + 1 < n_sup)
                def _():
                    fire(j + 1, 1 - b)

                drain(j, b)

                for p in range(2):

                    @pl.when(((j // 2) % 2 == p) & (j % 2 == 0) & (j >= 4))
                    def _():
                        wait_pair(j // 2 - 2, p)

                for p in range(2):

                    @pl.when((j // 2) % 2 == p)
                    def _():
                        @plsc.parallel_loop(0, SUP, unroll=4)
                        def _row(r):
                            for k in range(D_MODEL // LANES):
                                sl = pl.ds(k * LANES, LANES)
                                obs[p][b * SUP + r, sl] = (
                                    qbs[b][r, sl] * SCALE + b_regs[k]
                                )

                        @pl.when(j % 2 == 1)
                        def _():
                            fire_pair(j // 2, p)

    wait_pair(n_pair - 2, (n_pair - 2) % 2)
    wait_pair(n_pair - 1, (n_pair - 1) % 2)


def kernel(tokens, embed_weights, embed_bias):
    n_tok = tokens.shape[0] * tokens.shape[1]
    rows_per_w = n_tok // NW
    n_chunks = rows_per_w // CHUNK
    toks3d = tokens.reshape(NW, n_chunks, CHUNK)

    mesh = plsc.VectorSubcoreMesh(
        core_axis_name="c", subcore_axis_name="s", num_cores=NC, num_subcores=NS
    )
    run = pl.kernel(
        _body,
        out_type=jax.ShapeDtypeStruct((n_tok, D_MODEL), jnp.float32),
        mesh=mesh,
        scratch_types=[
            pltpu.VMEM((n_chunks, CHUNK), jnp.int32),
            pltpu.VMEM((SUP, D_MODEL), jnp.float32),
            pltpu.VMEM((SUP, D_MODEL), jnp.float32),
            pltpu.VMEM((2 * SUP, D_MODEL), jnp.float32),
            pltpu.VMEM((2 * SUP, D_MODEL), jnp.float32),
            pltpu.VMEM((D_MODEL,), jnp.float32),
            pltpu.SemaphoreType.DMA,
            pltpu.SemaphoreType.DMA,
            pltpu.SemaphoreType.DMA,
            pltpu.SemaphoreType.DMA,
        ],
        compiler_params=pltpu.CompilerParams(use_tc_tiling_on_sc=False),
    )
    out = run(embed_weights, toks3d, embed_bias)
    return out.reshape(tokens.shape[0], tokens.shape[1], D_MODEL)
